# Initial kernel scaffold; baseline (speedup 1.0000x reference)
import jax
import jax.numpy as jnp
from jax.experimental import pallas as pl


def kernel(x, emb_weight):
    raise NotImplementedError("write your pallas kernel here")


# speedup vs baseline: 1.3595x; 1.3595x over previous
import functools

import jax
import jax.numpy as jnp
from jax import lax
from jax.experimental import pallas as pl
from jax.experimental.pallas import tpu as pltpu
from jax.experimental.pallas import tpu_sc as plsc

B = 16384
D = 32
K = 8192
BM = 256


def _tc_body(x_ref, et_ref, idx_ref, loss_ref):
    i = pl.program_id(0)
    x = x_ref[...]
    et = et_ref[...]
    a = jnp.sum(x * x, axis=1, keepdims=True)
    c = jnp.sum(et * et, axis=0, keepdims=True)
    b = jnp.dot(x, et, preferred_element_type=jnp.float32)
    d = a - 2.0 * b + c
    H = K // 2
    d0 = d[:, :H]
    d1 = d[:, H:]
    m0 = jnp.min(d0, axis=1, keepdims=True)
    m1 = jnp.min(d1, axis=1, keepdims=True)
    iota = lax.broadcasted_iota(jnp.int32, d0.shape, 1)
    i0 = jnp.min(jnp.where(d0 == m0, iota, H), axis=1, keepdims=True)
    i1 = jnp.min(jnp.where(d1 == m1, iota, H), axis=1, keepdims=True) + H
    m0bf = m0.astype(jnp.bfloat16).astype(jnp.float32)
    pick0 = m0bf <= m1
    idx_ref[...] = jnp.where(pick0, i0, i1)

    @pl.when(i == 0)
    def _init():
        loss_ref[0, 0] = 0.0

    loss_ref[0, 0] += jnp.sum(jnp.where(pick0, m0, m1))


def _tc_argmin(x, et):
    grid = B // BM
    return pl.pallas_call(
        _tc_body,
        grid=(grid,),
        in_specs=[
            pl.BlockSpec((BM, D), lambda i: (i, 0)),
            pl.BlockSpec((D, K), lambda i: (0, 0)),
        ],
        out_specs=[
            pl.BlockSpec((BM, 1), lambda i: (i, 0)),
            pl.BlockSpec(memory_space=pltpu.SMEM),
        ],
        out_shape=[
            jax.ShapeDtypeStruct((B, 1), jnp.int32),
            jax.ShapeDtypeStruct((1, 1), jnp.float32),
        ],
    )(x, et)


def _make_sc_gather():
    info = plsc.get_sparse_core_info()
    nw = info.num_cores * info.num_subcores
    rows_per_w = B // nw
    chunks = rows_per_w // 128

    mesh = plsc.VectorSubcoreMesh(core_axis_name="c", subcore_axis_name="s")

    @functools.partial(
        pl.kernel,
        mesh=mesh,
        out_type=jax.ShapeDtypeStruct((B // 128, 128, 128), jnp.float32),
        scratch_types=[
            pltpu.VMEM((chunks, 128), jnp.int32),
            pltpu.VMEM((chunks, 128, 128), jnp.float32),
            pltpu.SemaphoreType.DMA,
        ],
    )
    def gather(table_hbm, idx_hbm, out_hbm, idx_v, rows_v, sem):
        wid = lax.axis_index("s") * info.num_cores + lax.axis_index("c")
        base = wid * chunks
        pltpu.sync_copy(idx_hbm.at[pl.ds(base, chunks)], idx_v)
        copies = []
        for j in range(chunks):
            copies.append(
                pltpu.async_copy(table_hbm.at[idx_v.at[j]], rows_v.at[j], sem))
        for c in copies:
            c.wait()
        pltpu.sync_copy(rows_v, out_hbm.at[pl.ds(base, chunks)])

    return gather


_sc_gather = None


def kernel(x, emb_weight):
    global _sc_gather
    if _sc_gather is None:
        _sc_gather = _make_sc_gather()
    idx2d, loss_sum = _tc_argmin(x, emb_weight.T)
    indices = idx2d.reshape(B)
    idx_grid = idx2d.reshape(B // 128, 128)
    table = jnp.pad(emb_weight, ((0, 0), (0, 128 - D)))
    quantized = _sc_gather(table, idx_grid).reshape(B, 128)[:, :D]
    loss = loss_sum[0, 0] * (1.25 / x.size)
    return quantized, indices, loss


# speedup vs baseline: 1.4317x; 1.0531x over previous
import functools

import jax
import jax.numpy as jnp
from jax import lax
from jax.experimental import pallas as pl
from jax.experimental.pallas import tpu as pltpu
from jax.experimental.pallas import tpu_sc as plsc

B = 16384
D = 32
K = 8192
BM = 512


def _tc_body(x_ref, et_ref, idx_ref, loss_ref):
    i = pl.program_id(0)
    x = x_ref[...]
    et = et_ref[...]
    a = jnp.sum(x * x, axis=1, keepdims=True)
    c = jnp.sum(et * et, axis=0, keepdims=True)
    b = jnp.dot(x, et, preferred_element_type=jnp.float32)
    d = a - 2.0 * b + c
    H = K // 2
    d0 = d[:, :H]
    d1 = d[:, H:]
    m0 = jnp.min(d0, axis=1, keepdims=True)
    m1 = jnp.min(d1, axis=1, keepdims=True)
    iota = lax.broadcasted_iota(jnp.int32, d0.shape, 1)
    i0 = jnp.min(jnp.where(d0 == m0, iota, H), axis=1, keepdims=True)
    i1 = jnp.min(jnp.where(d1 == m1, iota, H), axis=1, keepdims=True) + H
    m0bf = m0.astype(jnp.bfloat16).astype(jnp.float32)
    pick0 = m0bf <= m1
    idx_ref[...] = jnp.where(pick0, i0, i1)

    @pl.when(i == 0)
    def _init():
        loss_ref[0, 0] = 0.0

    loss_ref[0, 0] += jnp.sum(jnp.where(pick0, m0, m1))


def _tc_argmin(x, et):
    grid = B // BM
    return pl.pallas_call(
        _tc_body,
        grid=(grid,),
        in_specs=[
            pl.BlockSpec((BM, D), lambda i: (i, 0)),
            pl.BlockSpec((D, K), lambda i: (0, 0)),
        ],
        out_specs=[
            pl.BlockSpec((BM, 1), lambda i: (i, 0)),
            pl.BlockSpec(memory_space=pltpu.SMEM),
        ],
        out_shape=[
            jax.ShapeDtypeStruct((B, 1), jnp.int32),
            jax.ShapeDtypeStruct((1, 1), jnp.float32),
        ],
    )(x, et)


def _make_sc_gather():
    info = plsc.get_sparse_core_info()
    nw = info.num_cores * info.num_subcores
    rows_per_w = B // nw
    chunks = rows_per_w // 128

    mesh = plsc.VectorSubcoreMesh(core_axis_name="c", subcore_axis_name="s")

    @functools.partial(
        pl.kernel,
        mesh=mesh,
        out_type=jax.ShapeDtypeStruct((B // 128, 128, 128), jnp.float32),
        scratch_types=[
            pltpu.VMEM((chunks, 128), jnp.int32),
            pltpu.VMEM((chunks, 128, 128), jnp.float32),
            pltpu.SemaphoreType.DMA,
        ],
    )
    def gather(table_hbm, idx_hbm, out_hbm, idx_v, rows_v, sem):
        wid = lax.axis_index("s") * info.num_cores + lax.axis_index("c")
        base = wid * chunks
        pltpu.sync_copy(idx_hbm.at[pl.ds(base, chunks)], idx_v)
        copies = []
        for j in range(chunks):
            copies.append(
                pltpu.async_copy(table_hbm.at[idx_v.at[j]], rows_v.at[j], sem))
        for c in copies:
            c.wait()
        pltpu.sync_copy(rows_v, out_hbm.at[pl.ds(base, chunks)])

    return gather


_sc_gather = None


def kernel(x, emb_weight):
    global _sc_gather
    if _sc_gather is None:
        _sc_gather = _make_sc_gather()
    idx2d, loss_sum = _tc_argmin(x, emb_weight.T)
    indices = idx2d.reshape(B)
    idx_grid = idx2d.reshape(B // 128, 128)
    table = jnp.pad(emb_weight, ((0, 0), (0, 128 - D)))
    quantized = _sc_gather(table, idx_grid).reshape(B, 128)[:, :D]
    loss = loss_sum[0, 0] * (1.25 / x.size)
    return quantized, indices, loss


# speedup vs baseline: 1.4835x; 1.0362x over previous
import functools

import jax
import jax.numpy as jnp
from jax import lax
from jax.experimental import pallas as pl
from jax.experimental.pallas import tpu as pltpu
from jax.experimental.pallas import tpu_sc as plsc

B = 16384
D = 32
K = 8192
BM = 1024


def _tc_body(x_ref, et_ref, idx_ref, loss_ref):
    i = pl.program_id(0)
    x = x_ref[...]
    et = et_ref[...]
    a = jnp.sum(x * x, axis=1, keepdims=True)
    c = jnp.sum(et * et, axis=0, keepdims=True)
    b = jnp.dot(x, et, preferred_element_type=jnp.float32)
    d = a - 2.0 * b + c
    H = K // 2
    d0 = d[:, :H]
    d1 = d[:, H:]
    m0 = jnp.min(d0, axis=1, keepdims=True)
    m1 = jnp.min(d1, axis=1, keepdims=True)
    iota = lax.broadcasted_iota(jnp.int32, d0.shape, 1)
    i0 = jnp.min(jnp.where(d0 == m0, iota, H), axis=1, keepdims=True)
    i1 = jnp.min(jnp.where(d1 == m1, iota, H), axis=1, keepdims=True) + H
    m0bf = m0.astype(jnp.bfloat16).astype(jnp.float32)
    pick0 = m0bf <= m1
    idx_ref[...] = jnp.where(pick0, i0, i1)

    @pl.when(i == 0)
    def _init():
        loss_ref[0, 0] = 0.0

    loss_ref[0, 0] += jnp.sum(jnp.where(pick0, m0, m1))


def _tc_argmin(x, et):
    grid = B // BM
    return pl.pallas_call(
        _tc_body,
        grid=(grid,),
        in_specs=[
            pl.BlockSpec((BM, D), lambda i: (i, 0)),
            pl.BlockSpec((D, K), lambda i: (0, 0)),
        ],
        out_specs=[
            pl.BlockSpec((BM, 1), lambda i: (i, 0)),
            pl.BlockSpec(memory_space=pltpu.SMEM),
        ],
        out_shape=[
            jax.ShapeDtypeStruct((B, 1), jnp.int32),
            jax.ShapeDtypeStruct((1, 1), jnp.float32),
        ],
    )(x, et)


def _make_sc_gather():
    info = plsc.get_sparse_core_info()
    nw = info.num_cores * info.num_subcores
    rows_per_w = B // nw
    chunks = rows_per_w // 128

    mesh = plsc.VectorSubcoreMesh(core_axis_name="c", subcore_axis_name="s")

    @functools.partial(
        pl.kernel,
        mesh=mesh,
        out_type=jax.ShapeDtypeStruct((B // 128, 128, 128), jnp.float32),
        scratch_types=[
            pltpu.VMEM((chunks, 128), jnp.int32),
            pltpu.VMEM((chunks, 128, 128), jnp.float32),
            pltpu.SemaphoreType.DMA,
        ],
    )
    def gather(table_hbm, idx_hbm, out_hbm, idx_v, rows_v, sem):
        wid = lax.axis_index("s") * info.num_cores + lax.axis_index("c")
        base = wid * chunks
        pltpu.sync_copy(idx_hbm.at[pl.ds(base, chunks)], idx_v)
        copies = []
        for j in range(chunks):
            copies.append(
                pltpu.async_copy(table_hbm.at[idx_v.at[j]], rows_v.at[j], sem))
        for c in copies:
            c.wait()
        pltpu.sync_copy(rows_v, out_hbm.at[pl.ds(base, chunks)])

    return gather


_sc_gather = None


def kernel(x, emb_weight):
    global _sc_gather
    if _sc_gather is None:
        _sc_gather = _make_sc_gather()
    idx2d, loss_sum = _tc_argmin(x, emb_weight.T)
    indices = idx2d.reshape(B)
    idx_grid = idx2d.reshape(B // 128, 128)
    table = jnp.pad(emb_weight, ((0, 0), (0, 128 - D)))
    quantized = _sc_gather(table, idx_grid).reshape(B, 128)[:, :D]
    loss = loss_sum[0, 0] * (1.25 / x.size)
    return quantized, indices, loss


# speedup vs baseline: 1.4876x; 1.0027x over previous
import functools

import jax
import jax.numpy as jnp
from jax import lax
from jax.experimental import pallas as pl
from jax.experimental.pallas import tpu as pltpu
from jax.experimental.pallas import tpu_sc as plsc

B = 16384
D = 32
K = 8192
BM = 2048


def _tc_body(x_ref, et_ref, idx_ref, loss_ref):
    i = pl.program_id(0)
    x = x_ref[...]
    et = et_ref[...]
    a = jnp.sum(x * x, axis=1, keepdims=True)
    c = jnp.sum(et * et, axis=0, keepdims=True)
    b = jnp.dot(x, et, preferred_element_type=jnp.float32)
    d = a - 2.0 * b + c
    H = K // 2
    d0 = d[:, :H]
    d1 = d[:, H:]
    m0 = jnp.min(d0, axis=1, keepdims=True)
    m1 = jnp.min(d1, axis=1, keepdims=True)
    iota = lax.broadcasted_iota(jnp.int32, d0.shape, 1)
    i0 = jnp.min(jnp.where(d0 == m0, iota, H), axis=1, keepdims=True)
    i1 = jnp.min(jnp.where(d1 == m1, iota, H), axis=1, keepdims=True) + H
    m0bf = m0.astype(jnp.bfloat16).astype(jnp.float32)
    pick0 = m0bf <= m1
    idx_ref[...] = jnp.where(pick0, i0, i1)

    @pl.when(i == 0)
    def _init():
        loss_ref[0, 0] = 0.0

    loss_ref[0, 0] += jnp.sum(jnp.where(pick0, m0, m1))


def _tc_argmin(x, et):
    grid = B // BM
    return pl.pallas_call(
        _tc_body,
        grid=(grid,),
        in_specs=[
            pl.BlockSpec((BM, D), lambda i: (i, 0)),
            pl.BlockSpec((D, K), lambda i: (0, 0)),
        ],
        out_specs=[
            pl.BlockSpec((BM, 1), lambda i: (i, 0)),
            pl.BlockSpec(memory_space=pltpu.SMEM),
        ],
        out_shape=[
            jax.ShapeDtypeStruct((B, 1), jnp.int32),
            jax.ShapeDtypeStruct((1, 1), jnp.float32),
        ],
    )(x, et)


def _make_sc_gather():
    info = plsc.get_sparse_core_info()
    nw = info.num_cores * info.num_subcores
    rows_per_w = B // nw
    chunks = rows_per_w // 128

    mesh = plsc.VectorSubcoreMesh(core_axis_name="c", subcore_axis_name="s")

    @functools.partial(
        pl.kernel,
        mesh=mesh,
        out_type=jax.ShapeDtypeStruct((B // 128, 128, 128), jnp.float32),
        scratch_types=[
            pltpu.VMEM((chunks, 128), jnp.int32),
            pltpu.VMEM((chunks, 128, 128), jnp.float32),
            pltpu.SemaphoreType.DMA,
        ],
    )
    def gather(table_hbm, idx_hbm, out_hbm, idx_v, rows_v, sem):
        wid = lax.axis_index("s") * info.num_cores + lax.axis_index("c")
        base = wid * chunks
        pltpu.sync_copy(idx_hbm.at[pl.ds(base, chunks)], idx_v)
        copies = []
        for j in range(chunks):
            copies.append(
                pltpu.async_copy(table_hbm.at[idx_v.at[j]], rows_v.at[j], sem))
        for c in copies:
            c.wait()
        pltpu.sync_copy(rows_v, out_hbm.at[pl.ds(base, chunks)])

    return gather


_sc_gather = None


def kernel(x, emb_weight):
    global _sc_gather
    if _sc_gather is None:
        _sc_gather = _make_sc_gather()
    idx2d, loss_sum = _tc_argmin(x, emb_weight.T)
    indices = idx2d.reshape(B)
    idx_grid = idx2d.reshape(B // 128, 128)
    table = jnp.pad(emb_weight, ((0, 0), (0, 128 - D)))
    quantized = _sc_gather(table, idx_grid).reshape(B, 128)[:, :D]
    loss = loss_sum[0, 0] * (1.25 / x.size)
    return quantized, indices, loss
